# baseline (speedup 1.0000x reference)
import functools

import jax
import jax.numpy as jnp
from jax.experimental import pallas as pl
from jax.experimental.pallas import tpu as pltpu

C_IN = 64
C_OUT = 64
N = 4096
T = 8192
BT = 256


def _fused_body(seq_ref, w0_ref, w1_ref, adj_ref, out_ref, at_ref, bt_ref):
    i = pl.program_id(0)

    @pl.when(i == 0)
    def _init():
        at_ref[...] = jax.lax.dot_general(
            seq_ref[...], w0_ref[...], (((0,), (1,)), ((), ())),
            preferred_element_type=jnp.float32)
        bt_ref[...] = jax.lax.dot_general(
            seq_ref[...], w1_ref[...], (((0,), (1,)), ((), ())),
            preferred_element_type=jnp.float32)

    blk = adj_ref[...]
    even = blk[:, :N]
    odd = blk[:, N:]

    ybt = jax.lax.dot_general(
        even, at_ref[...], (((1,), (0,)), ((), ())),
        preferred_element_type=jnp.float32)
    ybt += jax.lax.dot_general(
        odd, bt_ref[...], (((1,), (0,)), ((), ())),
        preferred_element_type=jnp.float32)

    contrib = jax.lax.dot_general(
        ybt, even, (((0,), (0,)), ((), ())),
        preferred_element_type=jnp.float32)

    @pl.when(i == 0)
    def _first():
        out_ref[...] = contrib

    @pl.when(i > 0)
    def _rest():
        out_ref[...] += contrib


@jax.jit
def kernel(seq, adj, W):
    adj2 = adj.reshape(T // 2, 2 * N)
    w0 = W[:, :, 0]
    w1 = W[:, :, 1]
    grid = (T // 2) // BT
    return pl.pallas_call(
        _fused_body,
        grid=(grid,),
        in_specs=[
            pl.BlockSpec((C_IN, N), lambda i: (0, 0)),
            pl.BlockSpec((C_OUT, C_IN), lambda i: (0, 0)),
            pl.BlockSpec((C_OUT, C_IN), lambda i: (0, 0)),
            pl.BlockSpec((BT, 2 * N), lambda i: (i, 0)),
        ],
        out_specs=pl.BlockSpec((C_OUT, N), lambda i: (0, 0)),
        out_shape=jax.ShapeDtypeStruct((C_OUT, N), jnp.float32),
        scratch_shapes=[
            pltpu.VMEM((N, C_OUT), jnp.float32),
            pltpu.VMEM((N, C_OUT), jnp.float32),
        ],
    )(seq, w0, w1, adj2)


# speedup vs baseline: 1.0019x; 1.0019x over previous
import jax
import jax.numpy as jnp
from jax.experimental import pallas as pl
from jax.experimental.pallas import tpu as pltpu

C_IN = 64
C_OUT = 64
N = 4096
T = 8192
BT = 512
NSTRIPS = 8
RS = BT // NSTRIPS


def _issue_block(adj_hbm, abuf, sems, block_idx, slot):
    for s in range(NSTRIPS):
        pltpu.make_async_copy(
            adj_hbm.at[pl.ds(block_idx * BT + s * RS, RS), :],
            abuf.at[slot, pl.ds(s * RS, RS), :],
            sems.at[slot, s],
        ).start()


def _wait_block(adj_hbm, abuf, sems, block_idx, slot):
    for s in range(NSTRIPS):
        pltpu.make_async_copy(
            adj_hbm.at[pl.ds(block_idx * BT + s * RS, RS), :],
            abuf.at[slot, pl.ds(s * RS, RS), :],
            sems.at[slot, s],
        ).wait()


def _fused_body(seq_ref, w0_ref, w1_ref, adj_hbm, out_ref, abuf, at_ref,
                bt_ref, sems):
    i = pl.program_id(0)
    nsteps = pl.num_programs(0)
    slot = jax.lax.rem(i, 2)

    @pl.when(i == 0)
    def _prologue():
        _issue_block(adj_hbm, abuf, sems, 0, 0)
        at_ref[...] = jax.lax.dot_general(
            seq_ref[...], w0_ref[...], (((0,), (1,)), ((), ())),
            preferred_element_type=jnp.float32)
        bt_ref[...] = jax.lax.dot_general(
            seq_ref[...], w1_ref[...], (((0,), (1,)), ((), ())),
            preferred_element_type=jnp.float32)

    _wait_block(adj_hbm, abuf, sems, i, slot)

    @pl.when(i + 1 < nsteps)
    def _prefetch():
        _issue_block(adj_hbm, abuf, sems, i + 1, 1 - slot)

    blk = abuf[slot]
    even = blk[:, :N]
    odd = blk[:, N:]

    ybt = jax.lax.dot_general(
        even, at_ref[...], (((1,), (0,)), ((), ())),
        preferred_element_type=jnp.float32)
    ybt += jax.lax.dot_general(
        odd, bt_ref[...], (((1,), (0,)), ((), ())),
        preferred_element_type=jnp.float32)

    contrib = jax.lax.dot_general(
        ybt, even, (((0,), (0,)), ((), ())),
        preferred_element_type=jnp.float32)

    @pl.when(i == 0)
    def _first():
        out_ref[...] = contrib

    @pl.when(i > 0)
    def _rest():
        out_ref[...] += contrib


@jax.jit
def kernel(seq, adj, W):
    adj2 = adj.reshape(T // 2, 2 * N)
    w0 = W[:, :, 0]
    w1 = W[:, :, 1]
    grid = (T // 2) // BT
    return pl.pallas_call(
        _fused_body,
        grid=(grid,),
        in_specs=[
            pl.BlockSpec((C_IN, N), lambda i: (0, 0)),
            pl.BlockSpec((C_OUT, C_IN), lambda i: (0, 0)),
            pl.BlockSpec((C_OUT, C_IN), lambda i: (0, 0)),
            pl.BlockSpec(memory_space=pltpu.MemorySpace.HBM),
        ],
        out_specs=pl.BlockSpec((C_OUT, N), lambda i: (0, 0)),
        out_shape=jax.ShapeDtypeStruct((C_OUT, N), jnp.float32),
        scratch_shapes=[
            pltpu.VMEM((2, BT, 2 * N), jnp.float32),
            pltpu.VMEM((N, C_OUT), jnp.float32),
            pltpu.VMEM((N, C_OUT), jnp.float32),
            pltpu.SemaphoreType.DMA((2, NSTRIPS)),
        ],
    )(seq, w0, w1, adj2)


# speedup vs baseline: 1.0132x; 1.0112x over previous
import jax
import jax.numpy as jnp
from jax.experimental import pallas as pl
from jax.experimental.pallas import tpu as pltpu

C_IN = 64
C_OUT = 64
N = 4096
T = 8192
BT = 512


def _fused_body(seq_ref, w0_ref, w1_ref, adj_ref, out_ref, at_ref, bt_ref):
    i = pl.program_id(0)

    @pl.when(i == 0)
    def _init():
        at_ref[...] = jax.lax.dot_general(
            seq_ref[...], w0_ref[...], (((0,), (1,)), ((), ())),
            preferred_element_type=jnp.float32)
        bt_ref[...] = jax.lax.dot_general(
            seq_ref[...], w1_ref[...], (((0,), (1,)), ((), ())),
            preferred_element_type=jnp.float32)

    blk = adj_ref[...]
    even = blk[:, :N]
    odd = blk[:, N:]

    ybt = jax.lax.dot_general(
        even, at_ref[...], (((1,), (0,)), ((), ())),
        preferred_element_type=jnp.float32)
    ybt += jax.lax.dot_general(
        odd, bt_ref[...], (((1,), (0,)), ((), ())),
        preferred_element_type=jnp.float32)

    contrib = jax.lax.dot_general(
        ybt, even, (((0,), (0,)), ((), ())),
        preferred_element_type=jnp.float32)

    @pl.when(i == 0)
    def _first():
        out_ref[...] = contrib

    @pl.when(i > 0)
    def _rest():
        out_ref[...] += contrib


@jax.jit
def kernel(seq, adj, W):
    adj2 = adj.reshape(T // 2, 2 * N)
    w0 = W[:, :, 0]
    w1 = W[:, :, 1]
    grid = (T // 2) // BT
    return pl.pallas_call(
        _fused_body,
        grid=(grid,),
        in_specs=[
            pl.BlockSpec((C_IN, N), lambda i: (0, 0)),
            pl.BlockSpec((C_OUT, C_IN), lambda i: (0, 0)),
            pl.BlockSpec((C_OUT, C_IN), lambda i: (0, 0)),
            pl.BlockSpec((BT, 2 * N), lambda i: (i, 0)),
        ],
        out_specs=pl.BlockSpec((C_OUT, N), lambda i: (0, 0)),
        out_shape=jax.ShapeDtypeStruct((C_OUT, N), jnp.float32),
        scratch_shapes=[
            pltpu.VMEM((N, C_OUT), jnp.float32),
            pltpu.VMEM((N, C_OUT), jnp.float32),
        ],
    )(seq, w0, w1, adj2)
